# baseline (speedup 1.0000x reference)
import functools

import jax
import jax.numpy as jnp
from jax import lax
from jax.experimental import pallas as pl
from jax.experimental.pallas import tpu as pltpu
from jax.experimental.pallas import tpu_sc as plsc

_N = 10000
_E = 320000
_D = 128
_NC = 2
_NS = 16
_NW = _NC * _NS
_EPT = _E // _NW
_C = 80
_NCHUNK = _EPT // _C
_NPAD = 10240
_RPT = _NPAD // _NS
_ZR = 128


_NB = 4


def _propagate_kernel(feat_hbm, src_hbm, dst_hbm, w_hbm, out_hbm,
                      sidx_b, didx_b, w_b, bufs, acc_sh,
                      isems, gsems, ssems):
    c = lax.axis_index("c")
    s = lax.axis_index("s")
    wid = s * _NC + c

    def istart(j, b):
        base = wid * _EPT + j * _C
        pltpu.async_copy(src_hbm.at[pl.ds(base, _C)], sidx_b[b], isems[b])
        pltpu.async_copy(dst_hbm.at[pl.ds(base, _C)], didx_b[b], isems[b])
        pltpu.async_copy(w_hbm.at[pl.ds(base, _C)], w_b[b], isems[b])

    def iwait(j, b):
        pltpu.make_async_copy(src_hbm.at[pl.ds(0, _C)], sidx_b[b],
                              isems[b]).wait()
        pltpu.make_async_copy(dst_hbm.at[pl.ds(0, _C)], didx_b[b],
                              isems[b]).wait()
        pltpu.make_async_copy(w_hbm.at[pl.ds(0, _C)], w_b[b],
                              isems[b]).wait()

    def gstart(j, b):
        pltpu.async_copy(feat_hbm.at[sidx_b[b]], bufs[b], gsems[b])

    def gwait(j, b):
        pltpu.make_async_copy(feat_hbm.at[sidx_b[b]], bufs[b],
                              gsems[b]).wait()

    def sstart(j, b):
        pltpu.async_copy(bufs[b], acc_sh.at[didx_b[b]], ssems[b], add=True)

    def swait(j, b):
        pltpu.make_async_copy(bufs[b], acc_sh.at[didx_b[b]],
                              ssems[b]).wait()

    def mult(j, b, unroll=1):
        buf = bufs[b]
        wref = w_b[b]

        def gb(g, carry):
            wv16 = wref[pl.ds(g * 16, 16)]
            for l in range(16):
                e = g * 16 + l
                wb = jnp.full((16,), wv16[l], jnp.float32)
                for d in range(_D // 16):
                    sl = pl.ds(d * 16, 16)
                    buf[e, sl] = buf[e, sl] * wb
            return carry

        lax.fori_loop(0, _C // 16, gb, 0, unroll=unroll)

    def step(j, b, sw=True, ist=True, gst=True, unroll=1):
        if sw:
            swait(j - 2, (b + 2) % _NB)
        if ist:
            istart(j + 2, (b + 2) % _NB)
        gwait(j, b)
        mult(j, b, unroll)
        sstart(j, b)
        if gst:
            iwait(j + 2, (b + 2) % _NB)
            gstart(j + 2, (b + 2) % _NB)

    istart(0, 0)
    istart(1, 1)
    zv = jnp.zeros((16,), jnp.float32)

    def zb(i, carry):
        r = i // 8
        j = i % 8
        bufs[3][r, pl.ds(j * 16, 16)] = zv
        return carry

    lax.fori_loop(0, _C * 8, zb, 0)
    for k in range(_RPT // _C):
        pltpu.async_copy(bufs[3], acc_sh.at[pl.ds(s * _RPT + k * _C, _C)],
                         ssems[3])
    iwait(0, 0)
    gstart(0, 0)
    iwait(1, 1)
    gstart(1, 1)
    for k in range(_RPT // _C):
        pltpu.make_async_copy(bufs[3], acc_sh.at[pl.ds(0, _C)],
                              ssems[3]).wait()
    plsc.subcore_barrier()
    step(0, 0, sw=False)
    step(1, 1, sw=False)
    step(2, 2)
    step(3, 3)

    def quad_body(i4, carry):
        j0 = i4 * _NB
        for k in range(_NB):
            step(j0 + k, k, unroll=_C // 16)
        return carry

    lax.fori_loop(1, (_NCHUNK - 5) // _NB, quad_body, 0)

    step(_NCHUNK - 5, (_NCHUNK - 5) % _NB)
    step(_NCHUNK - 4, (_NCHUNK - 4) % _NB)
    step(_NCHUNK - 3, (_NCHUNK - 3) % _NB)
    step(_NCHUNK - 2, (_NCHUNK - 2) % _NB, ist=False, gst=False)
    step(_NCHUNK - 1, (_NCHUNK - 1) % _NB, ist=False, gst=False)
    swait(_NCHUNK - 2, (_NCHUNK - 2) % _NB)
    swait(_NCHUNK - 1, (_NCHUNK - 1) % _NB)

    plsc.subcore_barrier()
    pltpu.sync_copy(acc_sh.at[pl.ds(s * _RPT, _RPT)],
                    out_hbm.at[c, pl.ds(s * _RPT, _RPT)])


@functools.partial(
    pl.kernel,
    out_type=jax.ShapeDtypeStruct((_NC, _NPAD, _D), jnp.float32),
    mesh=plsc.VectorSubcoreMesh(core_axis_name="c", subcore_axis_name="s"),
    scratch_types=[
        [pltpu.VMEM((_C,), jnp.int32)] * _NB,
        [pltpu.VMEM((_C,), jnp.int32)] * _NB,
        [pltpu.VMEM((_C,), jnp.float32)] * _NB,
        [pltpu.VMEM((_C, _D), jnp.float32)] * _NB,
        pltpu.VMEM_SHARED((_NPAD, _D), jnp.float32),
        [pltpu.SemaphoreType.DMA] * _NB,
        [pltpu.SemaphoreType.DMA] * _NB,
        [pltpu.SemaphoreType.DMA] * _NB,
    ],
)
def _propagate(feat_hbm, src_hbm, dst_hbm, w_hbm, out_hbm,
               sidx_b, didx_b, w_b, bufs, acc_sh, isems, gsems, ssems):
    _propagate_kernel(feat_hbm, src_hbm, dst_hbm, w_hbm, out_hbm,
                      sidx_b, didx_b, w_b, bufs, acc_sh,
                      isems, gsems, ssems)


_BN = 2000


def _dense1_body(p_ref, w_ref, b_ref, o_ref):
    a = p_ref[0] + p_ref[1]
    acc = jnp.dot(a, w_ref[...], preferred_element_type=jnp.float32)
    o_ref[...] = jnp.maximum(acc + b_ref[...], 0.0)


def _dense1(parts, W1, b1):
    return pl.pallas_call(
        _dense1_body,
        grid=(_N // _BN,),
        in_specs=[
            pl.BlockSpec((2, _BN, _D), lambda i: (0, i, 0)),
            pl.BlockSpec((_D, _D), lambda i: (0, 0)),
            pl.BlockSpec((1, _D), lambda i: (0, 0)),
        ],
        out_specs=pl.BlockSpec((_BN, _D), lambda i: (i, 0)),
        out_shape=jax.ShapeDtypeStruct((_N, _D), jnp.float32),
    )(parts, W1, b1.reshape(1, _D))


def _dense2_body(p_ref, w_ref, b_ref, mu_ref, z_ref, q_ref):
    a = p_ref[0] + p_ref[1]
    z = jnp.dot(a, w_ref[...], preferred_element_type=jnp.float32) + b_ref[...]
    z_ref[...] = z
    mu = mu_ref[...]
    zm = lax.dot_general(z, mu, (((1,), (1,)), ((), ())),
                         preferred_element_type=jnp.float32)
    z2 = jnp.sum(z * z, axis=1, keepdims=True)
    m2 = jnp.sum(mu * mu, axis=1)[None, :]
    d2 = (z2 - 2.0 * zm) + m2
    qv = 1.0 / (1.0 + d2 + 1e-8)
    qv = qv * qv
    q_ref[...] = qv / jnp.sum(qv, axis=1, keepdims=True)


def _dense2(parts, W2, b2, mu):
    k = mu.shape[0]
    dout = W2.shape[1]
    return pl.pallas_call(
        _dense2_body,
        grid=(_N // _BN,),
        in_specs=[
            pl.BlockSpec((2, _BN, _D), lambda i: (0, i, 0)),
            pl.BlockSpec((_D, dout), lambda i: (0, 0)),
            pl.BlockSpec((1, dout), lambda i: (0, 0)),
            pl.BlockSpec((k, dout), lambda i: (0, 0)),
        ],
        out_specs=[
            pl.BlockSpec((_BN, dout), lambda i: (i, 0)),
            pl.BlockSpec((_BN, k), lambda i: (i, 0)),
        ],
        out_shape=[
            jax.ShapeDtypeStruct((_N, dout), jnp.float32),
            jax.ShapeDtypeStruct((_N, k), jnp.float32),
        ],
    )(parts, W2, b2.reshape(1, dout), mu)


def kernel(x, edge_index, edge_weight, W1, b1, W2, b2, mu):
    src = edge_index[0]
    dst = edge_index[1]
    ew = edge_weight
    parts1 = _propagate(x, src, dst, ew)
    h = _dense1(parts1, W1, b1)
    parts2 = _propagate(h, src, dst, ew)
    z, q = _dense2(parts2, W2, b2, mu)
    return (z, q)


# speedup vs baseline: 1.3165x; 1.3165x over previous
import functools

import jax
import jax.numpy as jnp
from jax import lax
from jax.experimental import pallas as pl
from jax.experimental.pallas import tpu as pltpu
from jax.experimental.pallas import tpu_sc as plsc

_N = 10000
_E = 320000
_D = 128
_NC = 2
_NS = 16
_NW = _NC * _NS
_EPT = _E // _NW
_C = 80
_NCHUNK = _EPT // _C
_NPAD = 10240
_RPT = _NPAD // _NS
_ZR = 128


_NB = 4


def _propagate_kernel(feat_hbm, src_hbm, dst_hbm, w_hbm, out_hbm,
                      sidx_b, didx_b, w_b, bufs, acc_sh,
                      isems, gsems, ssems):
    c = lax.axis_index("c")
    s = lax.axis_index("s")
    wid = s * _NC + c

    def istart(j, b):
        base = wid * _EPT + j * _C
        pltpu.async_copy(src_hbm.at[pl.ds(base, _C)], sidx_b[b], isems[b])
        pltpu.async_copy(dst_hbm.at[pl.ds(base, _C)], didx_b[b], isems[b])
        pltpu.async_copy(w_hbm.at[pl.ds(base, _C)], w_b[b], isems[b])

    def iwait(j, b):
        pltpu.make_async_copy(src_hbm.at[pl.ds(0, _C)], sidx_b[b],
                              isems[b]).wait()
        pltpu.make_async_copy(dst_hbm.at[pl.ds(0, _C)], didx_b[b],
                              isems[b]).wait()
        pltpu.make_async_copy(w_hbm.at[pl.ds(0, _C)], w_b[b],
                              isems[b]).wait()

    def gstart(j, b):
        pltpu.async_copy(feat_hbm.at[sidx_b[b]], bufs[b], gsems[b])

    def gwait(j, b):
        pltpu.make_async_copy(feat_hbm.at[sidx_b[b]], bufs[b],
                              gsems[b]).wait()

    def sstart(j, b):
        pltpu.async_copy(bufs[b], acc_sh.at[didx_b[b]], ssems[b], add=True)

    def swait(j, b):
        pltpu.make_async_copy(bufs[b], acc_sh.at[didx_b[b]],
                              ssems[b]).wait()

    def mult(j, b):
        buf = bufs[b]
        wref = w_b[b]

        def gb(g, carry):
            wv16 = wref[pl.ds(g * 16, 16)]
            for l in range(16):
                e = g * 16 + l
                wb = jnp.full((16,), wv16[l], jnp.float32)
                for d in range(_D // 16):
                    sl = pl.ds(d * 16, 16)
                    buf[e, sl] = buf[e, sl] * wb
            return carry

        lax.fori_loop(0, _C // 16, gb, 0)

    def step(j, b, sw=True, ist=True, gst=True):
        if sw:
            swait(j - 2, (b + 2) % _NB)
        if ist:
            istart(j + 2, (b + 2) % _NB)
        gwait(j, b)
        mult(j, b)
        sstart(j, b)
        if gst:
            iwait(j + 2, (b + 2) % _NB)
            gstart(j + 2, (b + 2) % _NB)

    istart(0, 0)
    istart(1, 1)
    zv = jnp.zeros((16,), jnp.float32)

    def zb(i, carry):
        r = i // 8
        j = i % 8
        bufs[3][r, pl.ds(j * 16, 16)] = zv
        return carry

    lax.fori_loop(0, _C * 8, zb, 0)
    for k in range(_RPT // _C):
        pltpu.async_copy(bufs[3], acc_sh.at[pl.ds(s * _RPT + k * _C, _C)],
                         ssems[3])
    iwait(0, 0)
    gstart(0, 0)
    iwait(1, 1)
    gstart(1, 1)
    for k in range(_RPT // _C):
        pltpu.make_async_copy(bufs[3], acc_sh.at[pl.ds(0, _C)],
                              ssems[3]).wait()
    plsc.subcore_barrier()
    step(0, 0, sw=False)
    step(1, 1, sw=False)
    step(2, 2)
    step(3, 3)

    def quad_body(i4, carry):
        j0 = i4 * _NB
        for k in range(_NB):
            step(j0 + k, k)
        return carry

    lax.fori_loop(1, (_NCHUNK - 5) // _NB, quad_body, 0)

    step(_NCHUNK - 5, (_NCHUNK - 5) % _NB)
    step(_NCHUNK - 4, (_NCHUNK - 4) % _NB)
    step(_NCHUNK - 3, (_NCHUNK - 3) % _NB)
    step(_NCHUNK - 2, (_NCHUNK - 2) % _NB, ist=False, gst=False)
    step(_NCHUNK - 1, (_NCHUNK - 1) % _NB, ist=False, gst=False)
    swait(_NCHUNK - 2, (_NCHUNK - 2) % _NB)
    swait(_NCHUNK - 1, (_NCHUNK - 1) % _NB)

    plsc.subcore_barrier()
    pltpu.sync_copy(acc_sh.at[pl.ds(s * _RPT, _RPT)],
                    out_hbm.at[c, pl.ds(s * _RPT, _RPT)])


@functools.partial(
    pl.kernel,
    out_type=jax.ShapeDtypeStruct((_NC, _NPAD, _D), jnp.float32),
    mesh=plsc.VectorSubcoreMesh(core_axis_name="c", subcore_axis_name="s"),
    scratch_types=[
        [pltpu.VMEM((_C,), jnp.int32)] * _NB,
        [pltpu.VMEM((_C,), jnp.int32)] * _NB,
        [pltpu.VMEM((_C,), jnp.float32)] * _NB,
        [pltpu.VMEM((_C, _D), jnp.float32)] * _NB,
        pltpu.VMEM_SHARED((_NPAD, _D), jnp.float32),
        [pltpu.SemaphoreType.DMA] * _NB,
        [pltpu.SemaphoreType.DMA] * _NB,
        [pltpu.SemaphoreType.DMA] * _NB,
    ],
)
def _propagate(feat_hbm, src_hbm, dst_hbm, w_hbm, out_hbm,
               sidx_b, didx_b, w_b, bufs, acc_sh, isems, gsems, ssems):
    _propagate_kernel(feat_hbm, src_hbm, dst_hbm, w_hbm, out_hbm,
                      sidx_b, didx_b, w_b, bufs, acc_sh,
                      isems, gsems, ssems)


_BN = 2000


def _dense1_body(p_ref, w_ref, b_ref, o_ref):
    a = p_ref[0] + p_ref[1]
    acc = jnp.dot(a, w_ref[...], preferred_element_type=jnp.float32)
    o_ref[...] = jnp.maximum(acc + b_ref[...], 0.0)


def _dense1(parts, W1, b1):
    return pl.pallas_call(
        _dense1_body,
        grid=(_N // _BN,),
        in_specs=[
            pl.BlockSpec((2, _BN, _D), lambda i: (0, i, 0)),
            pl.BlockSpec((_D, _D), lambda i: (0, 0)),
            pl.BlockSpec((1, _D), lambda i: (0, 0)),
        ],
        out_specs=pl.BlockSpec((_BN, _D), lambda i: (i, 0)),
        out_shape=jax.ShapeDtypeStruct((_N, _D), jnp.float32),
    )(parts, W1, b1.reshape(1, _D))


def _dense2_body(p_ref, w_ref, b_ref, mu_ref, z_ref, q_ref):
    a = p_ref[0] + p_ref[1]
    z = jnp.dot(a, w_ref[...], preferred_element_type=jnp.float32) + b_ref[...]
    z_ref[...] = z
    mu = mu_ref[...]
    zm = lax.dot_general(z, mu, (((1,), (1,)), ((), ())),
                         preferred_element_type=jnp.float32)
    z2 = jnp.sum(z * z, axis=1, keepdims=True)
    m2 = jnp.sum(mu * mu, axis=1)[None, :]
    d2 = (z2 - 2.0 * zm) + m2
    qv = 1.0 / (1.0 + d2 + 1e-8)
    qv = qv * qv
    q_ref[...] = qv / jnp.sum(qv, axis=1, keepdims=True)


def _dense2(parts, W2, b2, mu):
    k = mu.shape[0]
    dout = W2.shape[1]
    return pl.pallas_call(
        _dense2_body,
        grid=(_N // _BN,),
        in_specs=[
            pl.BlockSpec((2, _BN, _D), lambda i: (0, i, 0)),
            pl.BlockSpec((_D, dout), lambda i: (0, 0)),
            pl.BlockSpec((1, dout), lambda i: (0, 0)),
            pl.BlockSpec((k, dout), lambda i: (0, 0)),
        ],
        out_specs=[
            pl.BlockSpec((_BN, dout), lambda i: (i, 0)),
            pl.BlockSpec((_BN, k), lambda i: (i, 0)),
        ],
        out_shape=[
            jax.ShapeDtypeStruct((_N, dout), jnp.float32),
            jax.ShapeDtypeStruct((_N, k), jnp.float32),
        ],
    )(parts, W2, b2.reshape(1, dout), mu)


def kernel(x, edge_index, edge_weight, W1, b1, W2, b2, mu):
    src = edge_index[0]
    dst = edge_index[1]
    ew = edge_weight
    parts1 = _propagate(x, src, dst, ew)
    h = _dense1(parts1, W1, b1)
    parts2 = _propagate(h, src, dst, ew)
    z, q = _dense2(parts2, W2, b2, mu)
    return (z, q)
